# baseline (speedup 1.0000x reference)
import functools

import jax
import jax.numpy as jnp
from jax import lax
from jax.experimental import pallas as pl
from jax.experimental.pallas import tpu as pltpu
from jax.experimental.pallas import tpu_sc as plsc

LOG_REG_C = 2.5e-05
LEAF_COEF = 10.0
NTRANS = 50000
N_PARENTS = 800000
N_STEMS = 400000
NUM_BLOCKS = 105

NC = 2
NS = 16
NW = NC * NS

BIN_CHUNK = 3200
NBINS_PAD = NS * BIN_CHUNK

ROW_BLK = 3200


ROW_CHUNK = 16000


def _rowsum_body(x_ref, o_ref):
    acc = jnp.exp(x_ref[0, 0, 0, :])
    for f in range(1, NUM_BLOCKS):
        acc += jnp.exp(x_ref[f, 0, 0, :])
    o_ref[0, 0, :] = acc


def _tc_rowsum(stem_out_s):
    n = stem_out_s.shape[0]
    nchunk = n // ROW_CHUNK
    x4 = stem_out_s.T.reshape(NUM_BLOCKS, nchunk, 1, ROW_CHUNK)
    out = pl.pallas_call(
        _rowsum_body,
        grid=(nchunk,),
        in_specs=[pl.BlockSpec((NUM_BLOCKS, 1, 1, ROW_CHUNK),
                               lambda i: (0, i, 0, 0))],
        out_specs=pl.BlockSpec((1, 1, ROW_CHUNK), lambda i: (i, 0, 0)),
        out_shape=jax.ShapeDtypeStruct((nchunk, 1, ROW_CHUNK), jnp.float32),
    )(x4)
    return out


@functools.cache
def _make_sc_scatter(n, win, do_exp):
    mesh = plsc.VectorSubcoreMesh(
        core_axis_name="c", subcore_axis_name="s",
        num_cores=NC, num_subcores=NS)
    npw = n // NW
    assert n % NW == 0 and win % 16 == 0 and win >= npw + 8
    assert (n - win) % 8 == 0

    @functools.partial(
        pl.kernel,
        out_type=jax.ShapeDtypeStruct((NC * NBINS_PAD,), jnp.float32),
        mesh=mesh,
        scratch_types=[
            pltpu.VMEM((win,), jnp.int32),
            pltpu.VMEM((win,), jnp.float32),
            pltpu.VMEM((BIN_CHUNK,), jnp.float32),
            pltpu.VMEM_SHARED((NBINS_PAD,), jnp.float32),
            pltpu.SemaphoreType.DMA,
        ],
    )
    def sc_scatter(idx_hbm, val_hbm, out_hbm, idx_v, val_v, zero_v, acc_sh, sem):
        c = lax.axis_index("c")
        s = lax.axis_index("s")
        wid = s * NC + c

        def zbody(i, carry):
            zero_v[pl.ds(i * 16, 16)] = jnp.zeros((16,), jnp.float32)
            return carry

        lax.fori_loop(0, BIN_CHUNK // 16, zbody, 0)
        pltpu.sync_copy(zero_v, acc_sh.at[pl.ds(s * BIN_CHUNK, BIN_CHUNK)])

        e0 = wid * npw
        s0 = jnp.minimum((e0 // 8) * 8, n - win)
        s0 = pl.multiple_of(s0, 8)
        lo = e0 - s0
        hi = lo + npw

        pltpu.sync_copy(idx_hbm.at[pl.ds(s0, win)], idx_v)
        pltpu.sync_copy(val_hbm.at[pl.ds(s0, win)], val_v)

        def ebody(i, carry):
            off = i * 16
            p = off + lax.iota(jnp.int32, 16)
            valid = jnp.logical_and(p >= lo, p < hi)
            v = val_v[pl.ds(off, 16)]
            if do_exp:
                v = jnp.exp(v)
            val_v[pl.ds(off, 16)] = jnp.where(valid, v, jnp.zeros_like(v))
            return carry

        lax.fori_loop(0, win // 16, ebody, 0)

        plsc.subcore_barrier()

        pltpu.async_copy(val_v, acc_sh.at[idx_v], sem, add=True).wait()

        plsc.subcore_barrier()
        pltpu.sync_copy(
            acc_sh.at[pl.ds(s * BIN_CHUNK, BIN_CHUNK)],
            out_hbm.at[pl.ds(c * NBINS_PAD + s * BIN_CHUNK, BIN_CHUNK)],
        )

    return sc_scatter


def _final_body(infl_ref, stems_ref, mol_ref, r_ref, d_ref,
                loss_ref, term_ref, flow_ref):
    exp_inflow = infl_ref[0:1, :NTRANS] + infl_ref[1:2, :NTRANS]
    inflow = jnp.log(exp_inflow + LOG_REG_C)
    d = d_ref[:]
    r = r_ref[:]
    exp_outflow = (stems_ref[0:1, :NTRANS] + stems_ref[1:2, :NTRANS]
                   + jnp.exp(mol_ref[:]))
    outflow_plus_r = jnp.log(LOG_REG_C + r + exp_outflow * (1.0 - d))
    losses = (inflow - outflow_plus_r) ** 2
    om = 1.0 - d
    term = jnp.sum(losses * d) / (jnp.sum(d) + 1e-20)
    flow = jnp.sum(losses * om) / (jnp.sum(om) + 1e-20)
    loss_ref[0, 0] = term * LEAF_COEF + flow
    term_ref[0, 0] = term
    flow_ref[0, 0] = flow


def _tc_final(infl, stems, mol, r, d):
    smem_spec = pl.BlockSpec(memory_space=pltpu.SMEM)
    scalar = jax.ShapeDtypeStruct((1, 1), jnp.float32)
    return pl.pallas_call(
        _final_body,
        out_shape=(scalar, scalar, scalar),
        out_specs=(smem_spec, smem_spec, smem_spec),
    )(infl, stems, mol, r, d)


def kernel(stem_out_s, mol_out_s, qsa_p, r, d, pb, stem_batch):
    pb1 = pb.astype(jnp.int32)
    sb1 = stem_batch.astype(jnp.int32)

    row_sum = _tc_rowsum(stem_out_s)

    infl = _make_sc_scatter(N_PARENTS, 25008, True)(
        pb1, qsa_p).reshape(NC, NBINS_PAD)
    stems = _make_sc_scatter(N_STEMS, 12512, False)(
        sb1, row_sum.reshape(N_STEMS,)).reshape(NC, NBINS_PAD)

    mol2 = mol_out_s.reshape(1, NTRANS)
    r2 = r.reshape(1, NTRANS)
    d2 = d.reshape(1, NTRANS)
    loss, term, flow = _tc_final(infl, stems, mol2, r2, d2)
    return (loss[0, 0], term[0, 0], flow[0, 0])


# speedup vs baseline: 1.4647x; 1.4647x over previous
import functools

import jax
import jax.numpy as jnp
from jax import lax
from jax.experimental import pallas as pl
from jax.experimental.pallas import tpu as pltpu
from jax.experimental.pallas import tpu_sc as plsc

LOG_REG_C = 2.5e-05
LEAF_COEF = 10.0
NTRANS = 50000
N_PARENTS = 800000
N_STEMS = 400000
NUM_BLOCKS = 105

NC = 2
NS = 16
NW = NC * NS

BIN_CHUNK = 3200
NBINS_PAD = NS * BIN_CHUNK

ROW_BLK = 3200


ROW_SUB = 8
ROW_LANE = 2000


def _rowsum_body(x_ref, o_ref):
    acc = jnp.exp(x_ref[0, 0])
    for f in range(1, NUM_BLOCKS):
        acc += jnp.exp(x_ref[f, 0])
    o_ref[0] = acc


def _tc_rowsum(stem_out_s):
    n = stem_out_s.shape[0]
    nchunk = n // (ROW_SUB * ROW_LANE)
    x4 = stem_out_s.T.reshape(NUM_BLOCKS, nchunk, ROW_SUB, ROW_LANE)
    out = pl.pallas_call(
        _rowsum_body,
        grid=(nchunk,),
        in_specs=[pl.BlockSpec((NUM_BLOCKS, 1, ROW_SUB, ROW_LANE),
                               lambda i: (0, i, 0, 0))],
        out_specs=pl.BlockSpec((1, ROW_SUB, ROW_LANE), lambda i: (i, 0, 0)),
        out_shape=jax.ShapeDtypeStruct((nchunk, ROW_SUB, ROW_LANE),
                                       jnp.float32),
    )(x4)
    return out


@functools.cache
def _make_sc_scatter(n, win, do_exp):
    mesh = plsc.VectorSubcoreMesh(
        core_axis_name="c", subcore_axis_name="s",
        num_cores=NC, num_subcores=NS)
    npw = n // NW
    assert n % NW == 0 and win % 16 == 0 and win >= npw + 8
    assert (n - win) % 8 == 0

    @functools.partial(
        pl.kernel,
        out_type=jax.ShapeDtypeStruct((NC * NBINS_PAD,), jnp.float32),
        mesh=mesh,
        scratch_types=[
            pltpu.VMEM((win,), jnp.int32),
            pltpu.VMEM((win,), jnp.float32),
            pltpu.VMEM((BIN_CHUNK,), jnp.float32),
            pltpu.VMEM_SHARED((NBINS_PAD,), jnp.float32),
            pltpu.SemaphoreType.DMA,
        ],
    )
    def sc_scatter(idx_hbm, val_hbm, out_hbm, idx_v, val_v, zero_v, acc_sh, sem):
        c = lax.axis_index("c")
        s = lax.axis_index("s")
        wid = s * NC + c

        def zbody(i, carry):
            zero_v[pl.ds(i * 16, 16)] = jnp.zeros((16,), jnp.float32)
            return carry

        lax.fori_loop(0, BIN_CHUNK // 16, zbody, 0)
        pltpu.sync_copy(zero_v, acc_sh.at[pl.ds(s * BIN_CHUNK, BIN_CHUNK)])

        e0 = wid * npw
        s0 = jnp.minimum((e0 // 8) * 8, n - win)
        s0 = pl.multiple_of(s0, 8)
        lo = e0 - s0
        hi = lo + npw

        pltpu.sync_copy(idx_hbm.at[pl.ds(s0, win)], idx_v)
        pltpu.sync_copy(val_hbm.at[pl.ds(s0, win)], val_v)

        def ebody(i, carry):
            off = i * 16
            p = off + lax.iota(jnp.int32, 16)
            valid = jnp.logical_and(p >= lo, p < hi)
            v = val_v[pl.ds(off, 16)]
            if do_exp:
                v = jnp.exp(v)
            val_v[pl.ds(off, 16)] = jnp.where(valid, v, jnp.zeros_like(v))
            return carry

        lax.fori_loop(0, win // 16, ebody, 0)

        plsc.subcore_barrier()

        pltpu.async_copy(val_v, acc_sh.at[idx_v], sem, add=True).wait()

        plsc.subcore_barrier()
        pltpu.sync_copy(
            acc_sh.at[pl.ds(s * BIN_CHUNK, BIN_CHUNK)],
            out_hbm.at[pl.ds(c * NBINS_PAD + s * BIN_CHUNK, BIN_CHUNK)],
        )

    return sc_scatter


def _final_body(infl_ref, stems_ref, mol_ref, r_ref, d_ref,
                loss_ref, term_ref, flow_ref):
    exp_inflow = infl_ref[0:1, :NTRANS] + infl_ref[1:2, :NTRANS]
    inflow = jnp.log(exp_inflow + LOG_REG_C)
    d = d_ref[:]
    r = r_ref[:]
    exp_outflow = (stems_ref[0:1, :NTRANS] + stems_ref[1:2, :NTRANS]
                   + jnp.exp(mol_ref[:]))
    outflow_plus_r = jnp.log(LOG_REG_C + r + exp_outflow * (1.0 - d))
    losses = (inflow - outflow_plus_r) ** 2
    om = 1.0 - d
    term = jnp.sum(losses * d) / (jnp.sum(d) + 1e-20)
    flow = jnp.sum(losses * om) / (jnp.sum(om) + 1e-20)
    loss_ref[0, 0] = term * LEAF_COEF + flow
    term_ref[0, 0] = term
    flow_ref[0, 0] = flow


def _tc_final(infl, stems, mol, r, d):
    smem_spec = pl.BlockSpec(memory_space=pltpu.SMEM)
    scalar = jax.ShapeDtypeStruct((1, 1), jnp.float32)
    return pl.pallas_call(
        _final_body,
        out_shape=(scalar, scalar, scalar),
        out_specs=(smem_spec, smem_spec, smem_spec),
    )(infl, stems, mol, r, d)


def kernel(stem_out_s, mol_out_s, qsa_p, r, d, pb, stem_batch):
    pb1 = pb.astype(jnp.int32)
    sb1 = stem_batch.astype(jnp.int32)

    row_sum = _tc_rowsum(stem_out_s)

    infl = _make_sc_scatter(N_PARENTS, 25008, True)(
        pb1, qsa_p).reshape(NC, NBINS_PAD)
    stems = _make_sc_scatter(N_STEMS, 12512, False)(
        sb1, row_sum.reshape(N_STEMS,)).reshape(NC, NBINS_PAD)

    mol2 = mol_out_s.reshape(1, NTRANS)
    r2 = r.reshape(1, NTRANS)
    d2 = d.reshape(1, NTRANS)
    loss, term, flow = _tc_final(infl, stems, mol2, r2, d2)
    return (loss[0, 0], term[0, 0], flow[0, 0])


# speedup vs baseline: 5.6444x; 3.8535x over previous
import functools

import jax
import jax.numpy as jnp
from jax import lax
from jax.experimental import pallas as pl
from jax.experimental.pallas import tpu as pltpu
from jax.experimental.pallas import tpu_sc as plsc

LOG_REG_C = 2.5e-05
LEAF_COEF = 10.0
NTRANS = 50000
N_PARENTS = 800000
N_STEMS = 400000
NUM_BLOCKS = 105

NC = 2
NS = 16
NW = NC * NS

BIN_CHUNK = 3200
NBINS_PAD = NS * BIN_CHUNK

ROW_BLK = 3200


ROW_CHUNK = 16000


def _rowsum_body(x_ref, o_ref):
    o_ref[0, 0, :] = jnp.sum(jnp.exp(x_ref[:]), axis=0)


def _tc_rowsum(stem_out_s):
    n = stem_out_s.shape[0]
    nchunk = n // ROW_CHUNK
    out = pl.pallas_call(
        _rowsum_body,
        grid=(nchunk,),
        in_specs=[pl.BlockSpec((NUM_BLOCKS, ROW_CHUNK), lambda i: (0, i))],
        out_specs=pl.BlockSpec((1, 1, ROW_CHUNK), lambda i: (i, 0, 0)),
        out_shape=jax.ShapeDtypeStruct((nchunk, 1, ROW_CHUNK), jnp.float32),
    )(stem_out_s.T)
    return out


@functools.cache
def _make_sc_scatter(n, win, do_exp):
    mesh = plsc.VectorSubcoreMesh(
        core_axis_name="c", subcore_axis_name="s",
        num_cores=NC, num_subcores=NS)
    npw = n // NW
    assert n % NW == 0 and win % 16 == 0 and win >= npw + 8
    assert (n - win) % 8 == 0

    @functools.partial(
        pl.kernel,
        out_type=jax.ShapeDtypeStruct((NC * NBINS_PAD,), jnp.float32),
        mesh=mesh,
        scratch_types=[
            pltpu.VMEM((win,), jnp.int32),
            pltpu.VMEM((win,), jnp.float32),
            pltpu.VMEM((BIN_CHUNK,), jnp.float32),
            pltpu.VMEM_SHARED((NBINS_PAD,), jnp.float32),
            pltpu.SemaphoreType.DMA,
        ],
    )
    def sc_scatter(idx_hbm, val_hbm, out_hbm, idx_v, val_v, zero_v, acc_sh, sem):
        c = lax.axis_index("c")
        s = lax.axis_index("s")
        wid = s * NC + c

        def zbody(i, carry):
            zero_v[pl.ds(i * 16, 16)] = jnp.zeros((16,), jnp.float32)
            return carry

        lax.fori_loop(0, BIN_CHUNK // 16, zbody, 0)
        pltpu.sync_copy(zero_v, acc_sh.at[pl.ds(s * BIN_CHUNK, BIN_CHUNK)])

        e0 = wid * npw
        s0 = jnp.minimum((e0 // 8) * 8, n - win)
        s0 = pl.multiple_of(s0, 8)
        lo = e0 - s0
        hi = lo + npw

        pltpu.sync_copy(idx_hbm.at[pl.ds(s0, win)], idx_v)
        pltpu.sync_copy(val_hbm.at[pl.ds(s0, win)], val_v)

        def ebody(i, carry):
            off = i * 16
            p = off + lax.iota(jnp.int32, 16)
            valid = jnp.logical_and(p >= lo, p < hi)
            v = val_v[pl.ds(off, 16)]
            if do_exp:
                v = jnp.exp(v)
            val_v[pl.ds(off, 16)] = jnp.where(valid, v, jnp.zeros_like(v))
            return carry

        lax.fori_loop(0, win // 16, ebody, 0)

        plsc.subcore_barrier()

        pltpu.async_copy(val_v, acc_sh.at[idx_v], sem, add=True).wait()

        plsc.subcore_barrier()
        pltpu.sync_copy(
            acc_sh.at[pl.ds(s * BIN_CHUNK, BIN_CHUNK)],
            out_hbm.at[pl.ds(c * NBINS_PAD + s * BIN_CHUNK, BIN_CHUNK)],
        )

    return sc_scatter


def _final_body(infl_ref, stems_ref, mol_ref, r_ref, d_ref,
                loss_ref, term_ref, flow_ref):
    exp_inflow = infl_ref[0:1, :NTRANS] + infl_ref[1:2, :NTRANS]
    inflow = jnp.log(exp_inflow + LOG_REG_C)
    d = d_ref[:]
    r = r_ref[:]
    exp_outflow = (stems_ref[0:1, :NTRANS] + stems_ref[1:2, :NTRANS]
                   + jnp.exp(mol_ref[:]))
    outflow_plus_r = jnp.log(LOG_REG_C + r + exp_outflow * (1.0 - d))
    losses = (inflow - outflow_plus_r) ** 2
    om = 1.0 - d
    term = jnp.sum(losses * d) / (jnp.sum(d) + 1e-20)
    flow = jnp.sum(losses * om) / (jnp.sum(om) + 1e-20)
    loss_ref[0, 0] = term * LEAF_COEF + flow
    term_ref[0, 0] = term
    flow_ref[0, 0] = flow


def _tc_final(infl, stems, mol, r, d):
    smem_spec = pl.BlockSpec(memory_space=pltpu.SMEM)
    scalar = jax.ShapeDtypeStruct((1, 1), jnp.float32)
    return pl.pallas_call(
        _final_body,
        out_shape=(scalar, scalar, scalar),
        out_specs=(smem_spec, smem_spec, smem_spec),
    )(infl, stems, mol, r, d)


def kernel(stem_out_s, mol_out_s, qsa_p, r, d, pb, stem_batch):
    pb1 = pb.astype(jnp.int32)
    sb1 = stem_batch.astype(jnp.int32)

    infl = _make_sc_scatter(N_PARENTS, 25008, True)(
        pb1, qsa_p).reshape(NC, NBINS_PAD)

    row_sum = _tc_rowsum(stem_out_s)

    row_sum, infl = lax.optimization_barrier((row_sum, infl))

    stems = _make_sc_scatter(N_STEMS, 12512, False)(
        sb1, row_sum.reshape(N_STEMS,)).reshape(NC, NBINS_PAD)

    mol2 = mol_out_s.reshape(1, NTRANS)
    r2 = r.reshape(1, NTRANS)
    d2 = d.reshape(1, NTRANS)
    loss, term, flow = _tc_final(infl, stems, mol2, r2, d2)
    return (loss[0, 0], term[0, 0], flow[0, 0])


# speedup vs baseline: 6.2819x; 1.1129x over previous
import functools

import jax
import jax.numpy as jnp
from jax import lax
from jax.experimental import pallas as pl
from jax.experimental.pallas import tpu as pltpu
from jax.experimental.pallas import tpu_sc as plsc

LOG_REG_C = 2.5e-05
LEAF_COEF = 10.0
NTRANS = 50000
N_PARENTS = 800000
N_STEMS = 400000
NUM_BLOCKS = 105

NC = 2
NS = 16
NW = NC * NS

BIN_CHUNK = 3200
NBINS_PAD = NS * BIN_CHUNK

ROW_BLK = 3200


ROW_CHUNK = 16000


def _rowsum_body(x_ref, o_ref):
    i = pl.program_id(0)
    o_ref[pl.ds(i * ROW_CHUNK, ROW_CHUNK)] = jnp.sum(jnp.exp(x_ref[:]), axis=0)


def _tc_rowsum(stem_out_s):
    n = stem_out_s.shape[0]
    nchunk = n // ROW_CHUNK
    out = pl.pallas_call(
        _rowsum_body,
        grid=(nchunk,),
        in_specs=[pl.BlockSpec((NUM_BLOCKS, ROW_CHUNK), lambda i: (0, i))],
        out_specs=pl.BlockSpec((n,), lambda i: (0,)),
        out_shape=jax.ShapeDtypeStruct((n,), jnp.float32),
    )(stem_out_s.T)
    return out


@functools.cache
def _make_sc_scatter(n, win, do_exp):
    mesh = plsc.VectorSubcoreMesh(
        core_axis_name="c", subcore_axis_name="s",
        num_cores=NC, num_subcores=NS)
    npw = n // NW
    assert n % NW == 0 and win % 16 == 0 and win >= npw + 8
    assert (n - win) % 8 == 0

    @functools.partial(
        pl.kernel,
        out_type=jax.ShapeDtypeStruct((NC, NBINS_PAD), jnp.float32),
        mesh=mesh,
        scratch_types=[
            pltpu.VMEM((win,), jnp.int32),
            pltpu.VMEM((win,), jnp.float32),
            pltpu.VMEM((BIN_CHUNK,), jnp.float32),
            pltpu.VMEM_SHARED((NBINS_PAD,), jnp.float32),
            pltpu.SemaphoreType.DMA,
        ],
    )
    def sc_scatter(idx_hbm, val_hbm, out_hbm, idx_v, val_v, zero_v, acc_sh, sem):
        c = lax.axis_index("c")
        s = lax.axis_index("s")
        wid = s * NC + c

        def zbody(i, carry):
            zero_v[pl.ds(i * 16, 16)] = jnp.zeros((16,), jnp.float32)
            return carry

        lax.fori_loop(0, BIN_CHUNK // 16, zbody, 0)
        pltpu.sync_copy(zero_v, acc_sh.at[pl.ds(s * BIN_CHUNK, BIN_CHUNK)])

        e0 = wid * npw
        s0 = jnp.minimum((e0 // 8) * 8, n - win)
        s0 = pl.multiple_of(s0, 8)
        lo = e0 - s0
        hi = lo + npw

        pltpu.sync_copy(idx_hbm.at[pl.ds(s0, win)], idx_v)
        pltpu.sync_copy(val_hbm.at[pl.ds(s0, win)], val_v)

        def ebody(i, carry):
            off = i * 16
            p = off + lax.iota(jnp.int32, 16)
            valid = jnp.logical_and(p >= lo, p < hi)
            v = val_v[pl.ds(off, 16)]
            if do_exp:
                v = jnp.exp(v)
            val_v[pl.ds(off, 16)] = jnp.where(valid, v, jnp.zeros_like(v))
            return carry

        lax.fori_loop(0, win // 16, ebody, 0)

        plsc.subcore_barrier()

        pltpu.async_copy(val_v, acc_sh.at[idx_v], sem, add=True).wait()

        plsc.subcore_barrier()
        pltpu.sync_copy(
            acc_sh.at[pl.ds(s * BIN_CHUNK, BIN_CHUNK)],
            out_hbm.at[c, pl.ds(s * BIN_CHUNK, BIN_CHUNK)],
        )

    return sc_scatter


def _final_body(infl_ref, stems_ref, mol_ref, r_ref, d_ref,
                loss_ref, term_ref, flow_ref):
    exp_inflow = infl_ref[0:1, :NTRANS] + infl_ref[1:2, :NTRANS]
    inflow = jnp.log(exp_inflow + LOG_REG_C)
    d = d_ref[:]
    r = r_ref[:]
    exp_outflow = (stems_ref[0:1, :NTRANS] + stems_ref[1:2, :NTRANS]
                   + jnp.exp(mol_ref[:]))
    outflow_plus_r = jnp.log(LOG_REG_C + r + exp_outflow * (1.0 - d))
    losses = (inflow - outflow_plus_r) ** 2
    om = 1.0 - d
    term = jnp.sum(losses * d) / (jnp.sum(d) + 1e-20)
    flow = jnp.sum(losses * om) / (jnp.sum(om) + 1e-20)
    loss_ref[0, 0] = term * LEAF_COEF + flow
    term_ref[0, 0] = term
    flow_ref[0, 0] = flow


def _tc_final(infl, stems, mol, r, d):
    smem_spec = pl.BlockSpec(memory_space=pltpu.SMEM)
    scalar = jax.ShapeDtypeStruct((1, 1), jnp.float32)
    return pl.pallas_call(
        _final_body,
        out_shape=(scalar, scalar, scalar),
        out_specs=(smem_spec, smem_spec, smem_spec),
    )(infl, stems, mol, r, d)


def kernel(stem_out_s, mol_out_s, qsa_p, r, d, pb, stem_batch):
    pb1 = pb.astype(jnp.int32)
    sb1 = stem_batch.astype(jnp.int32)

    infl = _make_sc_scatter(N_PARENTS, 25008, True)(pb1, qsa_p)

    row_sum = _tc_rowsum(stem_out_s)

    row_sum, infl = lax.optimization_barrier((row_sum, infl))

    stems = _make_sc_scatter(N_STEMS, 12512, False)(sb1, row_sum)

    mol2 = mol_out_s.reshape(1, NTRANS)
    r2 = r.reshape(1, NTRANS)
    d2 = d.reshape(1, NTRANS)
    loss, term, flow = _tc_final(infl, stems, mol2, r2, d2)
    return (loss[0, 0], term[0, 0], flow[0, 0])


# speedup vs baseline: 6.2829x; 1.0002x over previous
import functools

import jax
import jax.numpy as jnp
from jax import lax
from jax.experimental import pallas as pl
from jax.experimental.pallas import tpu as pltpu
from jax.experimental.pallas import tpu_sc as plsc

LOG_REG_C = 2.5e-05
LEAF_COEF = 10.0
NTRANS = 50000
N_PARENTS = 800000
N_STEMS = 400000
NUM_BLOCKS = 105

NC = 2
NS = 16
NW = NC * NS

BIN_CHUNK = 3200
NBINS_PAD = NS * BIN_CHUNK

ROW_CHUNK = 16000


def _rowsum_body(x_ref, o_ref):
    i = pl.program_id(0)
    o_ref[pl.ds(i * ROW_CHUNK, ROW_CHUNK)] = jnp.sum(jnp.exp(x_ref[:]), axis=0)


def _tc_rowsum(stem_out_s):
    n = stem_out_s.shape[0]
    nchunk = n // ROW_CHUNK
    out = pl.pallas_call(
        _rowsum_body,
        grid=(nchunk,),
        in_specs=[pl.BlockSpec((NUM_BLOCKS, ROW_CHUNK), lambda i: (0, i))],
        out_specs=pl.BlockSpec((n,), lambda i: (0,)),
        out_shape=jax.ShapeDtypeStruct((n,), jnp.float32),
    )(stem_out_s.T)
    return out


@functools.cache
def _make_sc_scatter(n, win, do_exp):
    mesh = plsc.VectorSubcoreMesh(
        core_axis_name="c", subcore_axis_name="s",
        num_cores=NC, num_subcores=NS)
    npw = n // NW
    assert n % NW == 0 and win % 16 == 0 and win >= npw + 8
    assert (n - win) % 8 == 0

    @functools.partial(
        pl.kernel,
        out_type=jax.ShapeDtypeStruct((NC, NBINS_PAD), jnp.float32),
        mesh=mesh,
        scratch_types=[
            pltpu.VMEM((win,), jnp.int32),
            pltpu.VMEM((win,), jnp.float32),
            pltpu.VMEM((BIN_CHUNK,), jnp.float32),
            pltpu.VMEM_SHARED((NBINS_PAD,), jnp.float32),
            pltpu.SemaphoreType.DMA,
        ],
    )
    def sc_scatter(idx_hbm, val_hbm, out_hbm, idx_v, val_v, zero_v, acc_sh, sem):
        c = lax.axis_index("c")
        s = lax.axis_index("s")
        wid = s * NC + c

        def zbody(i, carry):
            zero_v[pl.ds(i * 16, 16)] = jnp.zeros((16,), jnp.float32)
            return carry

        lax.fori_loop(0, BIN_CHUNK // 16, zbody, 0)
        pltpu.sync_copy(zero_v, acc_sh.at[pl.ds(s * BIN_CHUNK, BIN_CHUNK)])

        e0 = wid * npw
        s0 = jnp.minimum((e0 // 8) * 8, n - win)
        s0 = pl.multiple_of(s0, 8)
        lo = e0 - s0
        hi = lo + npw

        pltpu.sync_copy(idx_hbm.at[pl.ds(s0, win)], idx_v)
        pltpu.sync_copy(val_hbm.at[pl.ds(s0, win)], val_v)

        def ebody(i, carry):
            off = i * 16
            p = off + lax.iota(jnp.int32, 16)
            valid = jnp.logical_and(p >= lo, p < hi)
            v = val_v[pl.ds(off, 16)]
            if do_exp:
                v = jnp.exp(v)
            val_v[pl.ds(off, 16)] = jnp.where(valid, v, jnp.zeros_like(v))
            return carry

        lax.fori_loop(0, win // 16, ebody, 0)

        plsc.subcore_barrier()

        pltpu.async_copy(val_v, acc_sh.at[idx_v], sem, add=True).wait()

        plsc.subcore_barrier()
        pltpu.sync_copy(
            acc_sh.at[pl.ds(s * BIN_CHUNK, BIN_CHUNK)],
            out_hbm.at[c, pl.ds(s * BIN_CHUNK, BIN_CHUNK)],
        )

    return sc_scatter


def _final_body(infl_ref, stems_ref, mol_ref, r_ref, d_ref,
                loss_ref, term_ref, flow_ref):
    exp_inflow = infl_ref[0:1, :NTRANS] + infl_ref[1:2, :NTRANS]
    inflow = jnp.log(exp_inflow + LOG_REG_C)
    d = d_ref[:]
    r = r_ref[:]
    exp_outflow = (stems_ref[0:1, :NTRANS] + stems_ref[1:2, :NTRANS]
                   + jnp.exp(mol_ref[:]))
    outflow_plus_r = jnp.log(LOG_REG_C + r + exp_outflow * (1.0 - d))
    losses = (inflow - outflow_plus_r) ** 2
    om = 1.0 - d
    term = jnp.sum(losses * d) / (jnp.sum(d) + 1e-20)
    flow = jnp.sum(losses * om) / (jnp.sum(om) + 1e-20)
    loss_ref[0, 0] = term * LEAF_COEF + flow
    term_ref[0, 0] = term
    flow_ref[0, 0] = flow


def _tc_final(infl, stems, mol, r, d):
    smem_spec = pl.BlockSpec(memory_space=pltpu.SMEM)
    scalar = jax.ShapeDtypeStruct((1, 1), jnp.float32)
    return pl.pallas_call(
        _final_body,
        out_shape=(scalar, scalar, scalar),
        out_specs=(smem_spec, smem_spec, smem_spec),
    )(infl, stems, mol, r, d)


def kernel(stem_out_s, mol_out_s, qsa_p, r, d, pb, stem_batch):
    pb1 = pb.astype(jnp.int32)
    sb1 = stem_batch.astype(jnp.int32)

    infl = _make_sc_scatter(N_PARENTS, 25008, True)(pb1, qsa_p)

    row_sum = _tc_rowsum(stem_out_s)

    row_sum, infl = lax.optimization_barrier((row_sum, infl))

    stems = _make_sc_scatter(N_STEMS, 12512, False)(sb1, row_sum)

    mol2 = mol_out_s.reshape(1, NTRANS)
    r2 = r.reshape(1, NTRANS)
    d2 = d.reshape(1, NTRANS)
    loss, term, flow = _tc_final(infl, stems, mol2, r2, d2)
    return (loss[0, 0], term[0, 0], flow[0, 0])
